# Initial kernel scaffold; baseline (speedup 1.0000x reference)
import jax
import jax.numpy as jnp
from jax.experimental import pallas as pl


def kernel(A, x, params):
    raise NotImplementedError("write your pallas kernel here")


# speedup vs baseline: 1.3212x; 1.3212x over previous
import functools
from typing import Any

import jax
import jax.numpy as jnp
from jax.experimental import pallas as pl

N = 4096
DIM = 48
K1 = int(0.8 * N)
K2 = int(0.7 * K1)
BM = 256
GRID = N // BM


def _fused_pass(A, arrays, scales, weights, parts, outspecs, score, c_out):
    n_arr = len(arrays)
    n_sc = len(scales)
    n_w = len(weights)

    def body(*refs):
        a_ref = refs[0]
        arr_refs = refs[1:1 + n_arr]
        sc_refs = refs[1 + n_arr:1 + n_arr + n_sc]
        w_refs = refs[1 + n_arr + n_sc:1 + n_arr + n_sc + n_w]
        out_ref = refs[-1]
        i = pl.program_id(0)

        group_vals = []
        for group in parts:
            acc = None
            for (aid, coef, sid) in group:
                v = arr_refs[aid][...]
                if sid is not None:
                    v = v * sc_refs[sid][...]
                if coef != 1.0:
                    v = v * coef
                acc = v if acc is None else acc + v
            group_vals.append(acc)

        a_blk = a_ref[...]
        raws = [jnp.dot(a_blk, gv, preferred_element_type=jnp.float32)
                for gv in group_vals]

        for spec in outspecs:
            y = None
            for (gi, wid) in spec["terms"]:
                t = raws[gi] if wid is None else jnp.dot(
                    raws[gi], w_refs[wid][...],
                    preferred_element_type=jnp.float32)
                y = t if y is None else y + t
            y = y + w_refs[spec["bias_id"]][...]
            if spec["act"] == "relu":
                y = jnp.maximum(y, 0.0)
            if spec["oscale_id"] is not None:
                y = y * sc_refs[spec["oscale_id"]][pl.ds(i * BM, BM), :]
            out_ref[:, spec["col"]:spec["col"] + spec["wout"]] = y

        if score is not None:
            hn = jnp.dot(raws[score["hn_group"]], w_refs[score["wg_id"]][...],
                         preferred_element_type=jnp.float32)
            hn = hn + w_refs[score["bg_id"]][...]
            if score["hn_col"] is not None:
                out_ref[:, score["hn_col"]:score["hn_col"] + DIM] = hn
            xs = jax.nn.sigmoid(hn)
            h = None
            for (aid, coef, sid) in parts[score["h_group"]]:
                v = arr_refs[aid][pl.ds(i * BM, BM), :]
                if sid is not None:
                    v = v * sc_refs[sid][pl.ds(i * BM, BM), :]
                if coef != 1.0:
                    v = v * coef
                h = v if h is None else h + v
            hw = jnp.dot(h, w_refs[score["wd_id"]][...],
                         preferred_element_type=jnp.float32)
            t = jnp.sum(hw * xs, axis=1, keepdims=True)
            t = t + w_refs[score["bd_id"]][...]
            out_ref[:, score["col"]:score["col"] + 1] = jax.nn.sigmoid(t)

    in_specs = [pl.BlockSpec((BM, N), lambda i: (i, 0))]
    for a in arrays:
        w = a.shape[1]
        in_specs.append(pl.BlockSpec((N, w), lambda i: (0, 0)))
    for _ in scales:
        in_specs.append(pl.BlockSpec((N, 1), lambda i: (0, 0)))
    for wgt in weights:
        in_specs.append(pl.BlockSpec(wgt.shape, lambda i: (0, 0)))

    return pl.pallas_call(
        body,
        grid=(GRID,),
        in_specs=in_specs,
        out_specs=pl.BlockSpec((BM, c_out), lambda i: (i, 0)),
        out_shape=jax.ShapeDtypeStruct((N, c_out), jnp.float32),
    )(A, *arrays, *scales, *weights)


def _linear(X, Wt, b, prelu_a=None):
    args = [X, Wt, b.reshape(1, -1)]
    if prelu_a is not None:
        args.append(prelu_a.reshape(1, 1))

    def body(*refs):
        x_ref, w_ref, b_ref = refs[:3]
        out_ref = refs[-1]
        y = jnp.dot(x_ref[...], w_ref[...],
                    preferred_element_type=jnp.float32) + b_ref[...]
        if prelu_a is not None:
            a_ref = refs[3]
            y = jnp.where(y >= 0.0, y, a_ref[0, 0] * y)
        out_ref[...] = y

    dout = Wt.shape[1]
    return pl.pallas_call(
        body,
        out_shape=jax.ShapeDtypeStruct((N, dout), jnp.float32),
    )(*args)


_BR = 128


def _rank_masks(scores, valid, k):
    s_col = scores.reshape(N, 1)
    s_row = scores.reshape(1, N)
    use_valid = valid is not None

    def body(*refs):
        if use_valid:
            (sc_ref, sr_ref, vr_ref, vc_ref, m_ref, sel_ref) = refs
        else:
            (sc_ref, sr_ref, m_ref, sel_ref) = refs
        i = pl.program_id(0)
        si = sc_ref[...]
        sj = sr_ref[...]
        ii = i * _BR + jax.lax.broadcasted_iota(jnp.int32, (_BR, N), 0)
        jj = jax.lax.broadcasted_iota(jnp.int32, (_BR, N), 1)
        gt = (sj > si).astype(jnp.float32)
        eq = jnp.where((sj == si) & (jj < ii), 1.0, 0.0)
        contrib = gt + eq
        if use_valid:
            contrib = contrib * vr_ref[...]
        rank = jnp.sum(contrib, axis=1, keepdims=True)
        sel = rank < float(k)
        if use_valid:
            sel = sel & (vc_ref[...] > 0.0)
        m_ref[...] = jnp.where(sel, si, 0.0)
        sel_ref[...] = jnp.where(sel, 1.0, 0.0)

    in_specs = [pl.BlockSpec((_BR, 1), lambda i: (i, 0)),
                pl.BlockSpec((1, N), lambda i: (0, 0))]
    args = [s_col, s_row]
    if use_valid:
        in_specs.append(pl.BlockSpec((1, N), lambda i: (0, 0)))
        in_specs.append(pl.BlockSpec((_BR, 1), lambda i: (i, 0)))
        args.append(valid.reshape(1, N))
        args.append(valid.reshape(N, 1))

    m, sel = pl.pallas_call(
        body,
        grid=(N // _BR,),
        in_specs=in_specs,
        out_specs=[pl.BlockSpec((_BR, 1), lambda i: (i, 0)),
                   pl.BlockSpec((_BR, 1), lambda i: (i, 0))],
        out_shape=[jax.ShapeDtypeStruct((N, 1), jnp.float32),
                   jax.ShapeDtypeStruct((N, 1), jnp.float32)],
    )(*args)
    return m, sel


def kernel(A, x, params: dict[str, Any]):
    p = params

    def wt(lin):
        return lin["W"].T

    def bias(lin):
        return lin["b"].reshape(1, -1)

    xw = _linear(x, wt(p["start_gcn_s1"]), jnp.zeros((DIM,), jnp.float32))
    x_s1 = _fused_pass(
        A, [xw], [], [bias(p["start_gcn_s1"])],
        parts=[[(0, 1.0, None)]],
        outspecs=[dict(terms=[(0, None)], bias_id=0, act=None,
                       oscale_id=None, col=0, wout=DIM)],
        score=None, c_out=DIM)

    is1 = p["is1"]
    h1 = _linear(x_s1, wt(is1["fc"]),
                 is1["fc"]["b"] + is1["fc"]["bias2"], is1["fc"]["a"])
    w2 = [wt(p["s1_l1"]), bias(p["s1_l1"]),
          wt(is1["gcn1"]), bias(is1["gcn1"]),
          is1["disc"]["W"][0], is1["disc"]["b"].reshape(1, 1)]
    pass2 = _fused_pass(
        A, [h1, x_s1], [], w2,
        parts=[[(0, 1.0, None)], [(1, 1.0, None)]],
        outspecs=[dict(terms=[(1, 0)], bias_id=1, act="relu",
                       oscale_id=None, col=0, wout=DIM)],
        score=dict(hn_group=0, wg_id=2, bg_id=3, wd_id=4, bd_id=5,
                   h_group=0, col=DIM, hn_col=None),
        c_out=DIM + 1)
    x_s1a = pass2[:, 0:DIM]
    scores1 = pass2[:, DIM]
    m1, sel1 = _rank_masks(scores1, None, K1)

    x_s2 = _fused_pass(
        A, [x_s1], [m1], [wt(p["start_gcn_s2"]), bias(p["start_gcn_s2"])],
        parts=[[(0, 1.0, 0)]],
        outspecs=[dict(terms=[(0, 0)], bias_id=1, act=None,
                       oscale_id=None, col=0, wout=DIM)],
        score=None, c_out=DIM)

    is2 = p["is2"]
    h1b = _linear(x_s2, wt(is2["fc"]),
                  is2["fc"]["b"] + is2["fc"]["bias2"], is2["fc"]["a"])
    w4 = [wt(p["s2_l1"]), bias(p["s2_l1"]),
          wt(is2["gcn1"]), bias(is2["gcn1"]),
          is2["disc"]["W"][0], is2["disc"]["b"].reshape(1, 1)]
    pass4 = _fused_pass(
        A, [h1b, x_s2], [sel1], w4,
        parts=[[(0, 1.0, 0)], [(1, 1.0, 0)]],
        outspecs=[dict(terms=[(1, 0)], bias_id=1, act="relu",
                       oscale_id=None, col=DIM, wout=DIM)],
        score=dict(hn_group=0, wg_id=2, bg_id=3, wd_id=4, bd_id=5,
                   h_group=0, col=2 * DIM, hn_col=0),
        c_out=2 * DIM + 1)
    xdown2 = pass4[:, 0:DIM]
    x_s2a = pass4[:, DIM:2 * DIM]
    scores2 = pass4[:, 2 * DIM]
    m2, sel2 = _rank_masks(scores2, sel1.reshape(N), K2)

    x_s3a = _fused_pass(
        A, [x_s2], [m2], [wt(p["s3_l1"]), bias(p["s3_l1"])],
        parts=[[(0, 1.0, 0)]],
        outspecs=[dict(terms=[(0, 0)], bias_id=1, act="relu",
                       oscale_id=None, col=0, wout=DIM)],
        score=None, c_out=DIM)

    def cross_pass(xs1, xs2, xs3, wp12, wu21, wp23, wu32):
        w = [wt(wp12), bias(wp12), wt(wu21), bias(wu21),
             wt(wp23), bias(wp23), wt(wu32), bias(wu32)]
        out = _fused_pass(
            A, [xs1, xs2, xs3], [sel1, sel2, m1, m2], w,
            parts=[[(0, 1.0, None)], [(1, 1.0, 0)], [(2, 1.0, 1)]],
            outspecs=[
                dict(terms=[(0, 0)], bias_id=1, act=None, oscale_id=2,
                     col=0, wout=DIM),
                dict(terms=[(1, 2)], bias_id=3, act=None, oscale_id=None,
                     col=DIM, wout=DIM),
                dict(terms=[(1, 4)], bias_id=5, act=None, oscale_id=3,
                     col=2 * DIM, wout=DIM),
                dict(terms=[(2, 6)], bias_id=7, act=None, oscale_id=None,
                     col=3 * DIM, wout=DIM),
            ],
            score=None, c_out=4 * DIM)
        return (out[:, 0:DIM], out[:, DIM:2 * DIM],
                out[:, 2 * DIM:3 * DIM], out[:, 3 * DIM:4 * DIM])

    x12, x21, x23, x32 = cross_pass(
        x_s1a, x_s2a, x_s3a,
        p["pool_s12_1"], p["unpool_s21_1"], p["pool_s23_1"], p["unpool_s32_1"])

    def tri_pass(parts, l1, l2, l3):
        w = [wt(l1), bias(l1), wt(l2), bias(l2), wt(l3), bias(l3)]
        out = _fused_pass(
            A, parts["arrays"], parts["scales"], w,
            parts=parts["groups"],
            outspecs=[
                dict(terms=[(0, 0)], bias_id=1, act="relu", oscale_id=None,
                     col=0, wout=DIM),
                dict(terms=[(1, 2)], bias_id=3, act="relu", oscale_id=None,
                     col=DIM, wout=DIM),
                dict(terms=[(2, 4)], bias_id=5, act="relu", oscale_id=None,
                     col=2 * DIM, wout=DIM),
            ],
            score=None, c_out=3 * DIM)
        return out[:, 0:DIM], out[:, DIM:2 * DIM], out[:, 2 * DIM:3 * DIM]

    p7 = dict(
        arrays=[x_s1a, x21, x_s1, x_s2a, x12, x32, x_s2, x_s3a, x23],
        scales=[sel1, sel2, m2],
        groups=[
            [(0, 1.0, None), (1, 1.0, None), (2, 1.0, None)],
            [(3, 1.0, 0), (4, 0.5, None), (5, 0.5, 0), (6, 1.0, 0)],
            [(7, 1.0, 1), (8, 1.0, None), (6, 1.0, 2)],
        ])
    x_s1b, x_s2b, x_s3b = tri_pass(p7, p["s1_l2"], p["s2_l2"], p["s3_l2"])

    x12b, x21b, x23b, x32b = cross_pass(
        x_s1b, x_s2b, x_s3b,
        p["pool_s12_2"], p["unpool_s21_2"], p["pool_s23_2"], p["unpool_s32_2"])

    p9 = dict(
        arrays=[x_s1b, x21b, x_s2b, x12b, x32b, x_s3b, x23b],
        scales=[sel1, sel2],
        groups=[
            [(0, 1.0, None), (1, 0.05, None)],
            [(2, 1.0, 0), (3, 0.025, None), (4, 0.025, 0)],
            [(5, 1.0, 1), (6, 0.05, None)],
        ])
    x_s1f, x_s2f, x_s3f = tri_pass(p9, p["s1_l3"], p["s2_l3"], p["s3_l3"])

    u32e = _fused_pass(
        A, [x_s3f], [sel2],
        [wt(p["unpool_s32_end"]), bias(p["unpool_s32_end"])],
        parts=[[(0, 1.0, 0)]],
        outspecs=[dict(terms=[(0, 0)], bias_id=1, act=None,
                       oscale_id=None, col=0, wout=DIM)],
        score=None, c_out=DIM)

    x_s2out = _fused_pass(
        A, [x_s2f, u32e, xdown2], [sel1],
        [wt(p["unpool_s21_end"]), bias(p["unpool_s21_end"])],
        parts=[[(0, 1.0, 0), (1, 1.0, 0), (2, 1.0, 0)]],
        outspecs=[dict(terms=[(0, 0)], bias_id=1, act=None,
                       oscale_id=None, col=0, wout=DIM)],
        score=None, c_out=DIM)

    wend = wt(p["end_gcn"])
    out = _fused_pass(
        A, [x_s1f, x_s2out], [],
        [wend[0:DIM, :], wend[DIM:2 * DIM, :], bias(p["end_gcn"])],
        parts=[[(0, 1.0, None)], [(1, 1.0, None)]],
        outspecs=[dict(terms=[(0, 0), (1, 1)], bias_id=2, act=None,
                       oscale_id=None, col=0, wout=256)],
        score=None, c_out=256)
    return out


# speedup vs baseline: 1.3622x; 1.0311x over previous
import functools
from typing import Any

import jax
import jax.numpy as jnp
from jax.experimental import pallas as pl

N = 4096
DIM = 48
K1 = int(0.8 * N)
K2 = int(0.7 * K1)
BM = 256
GRID = N // BM


def _fused_pass(A, arrays, scales, weights, parts, outspecs, score, c_out):
    n_arr = len(arrays)
    n_sc = len(scales)
    n_w = len(weights)

    def body(*refs):
        a_ref = refs[0]
        arr_refs = refs[1:1 + n_arr]
        sc_refs = refs[1 + n_arr:1 + n_arr + n_sc]
        w_refs = refs[1 + n_arr + n_sc:1 + n_arr + n_sc + n_w]
        out_ref = refs[-1]
        i = pl.program_id(0)

        group_vals = []
        for group in parts:
            acc = None
            for (aid, coef, sid) in group:
                v = arr_refs[aid][...]
                if sid is not None:
                    v = v * sc_refs[sid][...]
                if coef != 1.0:
                    v = v * coef
                acc = v if acc is None else acc + v
            group_vals.append(acc)

        a_blk = a_ref[...]
        raws = [jnp.dot(a_blk, gv.astype(a_blk.dtype),
                        preferred_element_type=jnp.float32)
                for gv in group_vals]

        for spec in outspecs:
            y = None
            for (gi, wid) in spec["terms"]:
                t = raws[gi] if wid is None else jnp.dot(
                    raws[gi], w_refs[wid][...],
                    preferred_element_type=jnp.float32)
                y = t if y is None else y + t
            y = y + w_refs[spec["bias_id"]][...]
            if spec["act"] == "relu":
                y = jnp.maximum(y, 0.0)
            if spec["oscale_id"] is not None:
                y = y * sc_refs[spec["oscale_id"]][pl.ds(i * BM, BM), :]
            out_ref[:, spec["col"]:spec["col"] + spec["wout"]] = y

        if score is not None:
            hn = jnp.dot(raws[score["hn_group"]], w_refs[score["wg_id"]][...],
                         preferred_element_type=jnp.float32)
            hn = hn + w_refs[score["bg_id"]][...]
            if score["hn_col"] is not None:
                out_ref[:, score["hn_col"]:score["hn_col"] + DIM] = hn
            xs = jax.nn.sigmoid(hn)
            h = None
            for (aid, coef, sid) in parts[score["h_group"]]:
                v = arr_refs[aid][pl.ds(i * BM, BM), :]
                if sid is not None:
                    v = v * sc_refs[sid][pl.ds(i * BM, BM), :]
                if coef != 1.0:
                    v = v * coef
                h = v if h is None else h + v
            hw = jnp.dot(h, w_refs[score["wd_id"]][...],
                         preferred_element_type=jnp.float32)
            t = jnp.sum(hw * xs, axis=1, keepdims=True)
            t = t + w_refs[score["bd_id"]][...]
            out_ref[:, score["col"]:score["col"] + 1] = jax.nn.sigmoid(t)

    in_specs = [pl.BlockSpec((BM, N), lambda i: (i, 0))]
    for a in arrays:
        w = a.shape[1]
        in_specs.append(pl.BlockSpec((N, w), lambda i: (0, 0)))
    for _ in scales:
        in_specs.append(pl.BlockSpec((N, 1), lambda i: (0, 0)))
    for wgt in weights:
        in_specs.append(pl.BlockSpec(wgt.shape, lambda i: (0, 0)))

    return pl.pallas_call(
        body,
        grid=(GRID,),
        in_specs=in_specs,
        out_specs=pl.BlockSpec((BM, c_out), lambda i: (i, 0)),
        out_shape=jax.ShapeDtypeStruct((N, c_out), jnp.float32),
    )(A, *arrays, *scales, *weights)


def _linear(X, Wt, b, prelu_a=None):
    args = [X, Wt, b.reshape(1, -1)]
    if prelu_a is not None:
        args.append(prelu_a.reshape(1, 1))

    def body(*refs):
        x_ref, w_ref, b_ref = refs[:3]
        out_ref = refs[-1]
        y = jnp.dot(x_ref[...], w_ref[...],
                    preferred_element_type=jnp.float32) + b_ref[...]
        if prelu_a is not None:
            a_ref = refs[3]
            y = jnp.where(y >= 0.0, y, a_ref[0, 0] * y)
        out_ref[...] = y

    dout = Wt.shape[1]
    return pl.pallas_call(
        body,
        out_shape=jax.ShapeDtypeStruct((N, dout), jnp.float32),
    )(*args)


_BR = 128


def _rank_masks(scores, valid, k):
    s_col = scores.reshape(N, 1)
    s_row = scores.reshape(1, N)
    use_valid = valid is not None

    def body(*refs):
        if use_valid:
            (sc_ref, sr_ref, vr_ref, vc_ref, m_ref, sel_ref) = refs
        else:
            (sc_ref, sr_ref, m_ref, sel_ref) = refs
        i = pl.program_id(0)
        si = sc_ref[...]
        sj = sr_ref[...]
        ii = i * _BR + jax.lax.broadcasted_iota(jnp.int32, (_BR, N), 0)
        jj = jax.lax.broadcasted_iota(jnp.int32, (_BR, N), 1)
        gt = (sj > si).astype(jnp.float32)
        eq = jnp.where((sj == si) & (jj < ii), 1.0, 0.0)
        contrib = gt + eq
        if use_valid:
            contrib = contrib * vr_ref[...]
        rank = jnp.sum(contrib, axis=1, keepdims=True)
        sel = rank < float(k)
        if use_valid:
            sel = sel & (vc_ref[...] > 0.0)
        m_ref[...] = jnp.where(sel, si, 0.0)
        sel_ref[...] = jnp.where(sel, 1.0, 0.0)

    in_specs = [pl.BlockSpec((_BR, 1), lambda i: (i, 0)),
                pl.BlockSpec((1, N), lambda i: (0, 0))]
    args = [s_col, s_row]
    if use_valid:
        in_specs.append(pl.BlockSpec((1, N), lambda i: (0, 0)))
        in_specs.append(pl.BlockSpec((_BR, 1), lambda i: (i, 0)))
        args.append(valid.reshape(1, N))
        args.append(valid.reshape(N, 1))

    m, sel = pl.pallas_call(
        body,
        grid=(N // _BR,),
        in_specs=in_specs,
        out_specs=[pl.BlockSpec((_BR, 1), lambda i: (i, 0)),
                   pl.BlockSpec((_BR, 1), lambda i: (i, 0))],
        out_shape=[jax.ShapeDtypeStruct((N, 1), jnp.float32),
                   jax.ShapeDtypeStruct((N, 1), jnp.float32)],
    )(*args)
    return m, sel


def kernel(A, x, params: dict[str, Any]):
    p = params
    A = A.astype(jnp.bfloat16)

    def wt(lin):
        return lin["W"].T

    def bias(lin):
        return lin["b"].reshape(1, -1)

    xw = _linear(x, wt(p["start_gcn_s1"]), jnp.zeros((DIM,), jnp.float32))
    x_s1 = _fused_pass(
        A, [xw], [], [bias(p["start_gcn_s1"])],
        parts=[[(0, 1.0, None)]],
        outspecs=[dict(terms=[(0, None)], bias_id=0, act=None,
                       oscale_id=None, col=0, wout=DIM)],
        score=None, c_out=DIM)

    is1 = p["is1"]
    h1 = _linear(x_s1, wt(is1["fc"]),
                 is1["fc"]["b"] + is1["fc"]["bias2"], is1["fc"]["a"])
    w2 = [wt(p["s1_l1"]), bias(p["s1_l1"]),
          wt(is1["gcn1"]), bias(is1["gcn1"]),
          is1["disc"]["W"][0], is1["disc"]["b"].reshape(1, 1)]
    pass2 = _fused_pass(
        A, [h1, x_s1], [], w2,
        parts=[[(0, 1.0, None)], [(1, 1.0, None)]],
        outspecs=[dict(terms=[(1, 0)], bias_id=1, act="relu",
                       oscale_id=None, col=0, wout=DIM)],
        score=dict(hn_group=0, wg_id=2, bg_id=3, wd_id=4, bd_id=5,
                   h_group=0, col=DIM, hn_col=None),
        c_out=DIM + 1)
    x_s1a = pass2[:, 0:DIM]
    scores1 = pass2[:, DIM]
    m1, sel1 = _rank_masks(scores1, None, K1)

    x_s2 = _fused_pass(
        A, [x_s1], [m1], [wt(p["start_gcn_s2"]), bias(p["start_gcn_s2"])],
        parts=[[(0, 1.0, 0)]],
        outspecs=[dict(terms=[(0, 0)], bias_id=1, act=None,
                       oscale_id=None, col=0, wout=DIM)],
        score=None, c_out=DIM)

    is2 = p["is2"]
    h1b = _linear(x_s2, wt(is2["fc"]),
                  is2["fc"]["b"] + is2["fc"]["bias2"], is2["fc"]["a"])
    w4 = [wt(p["s2_l1"]), bias(p["s2_l1"]),
          wt(is2["gcn1"]), bias(is2["gcn1"]),
          is2["disc"]["W"][0], is2["disc"]["b"].reshape(1, 1)]
    pass4 = _fused_pass(
        A, [h1b, x_s2], [sel1], w4,
        parts=[[(0, 1.0, 0)], [(1, 1.0, 0)]],
        outspecs=[dict(terms=[(1, 0)], bias_id=1, act="relu",
                       oscale_id=None, col=DIM, wout=DIM)],
        score=dict(hn_group=0, wg_id=2, bg_id=3, wd_id=4, bd_id=5,
                   h_group=0, col=2 * DIM, hn_col=0),
        c_out=2 * DIM + 1)
    xdown2 = pass4[:, 0:DIM]
    x_s2a = pass4[:, DIM:2 * DIM]
    scores2 = pass4[:, 2 * DIM]
    m2, sel2 = _rank_masks(scores2, sel1.reshape(N), K2)

    x_s3a = _fused_pass(
        A, [x_s2], [m2], [wt(p["s3_l1"]), bias(p["s3_l1"])],
        parts=[[(0, 1.0, 0)]],
        outspecs=[dict(terms=[(0, 0)], bias_id=1, act="relu",
                       oscale_id=None, col=0, wout=DIM)],
        score=None, c_out=DIM)

    def cross_pass(xs1, xs2, xs3, wp12, wu21, wp23, wu32):
        w = [wt(wp12), bias(wp12), wt(wu21), bias(wu21),
             wt(wp23), bias(wp23), wt(wu32), bias(wu32)]
        out = _fused_pass(
            A, [xs1, xs2, xs3], [sel1, sel2, m1, m2], w,
            parts=[[(0, 1.0, None)], [(1, 1.0, 0)], [(2, 1.0, 1)]],
            outspecs=[
                dict(terms=[(0, 0)], bias_id=1, act=None, oscale_id=2,
                     col=0, wout=DIM),
                dict(terms=[(1, 2)], bias_id=3, act=None, oscale_id=None,
                     col=DIM, wout=DIM),
                dict(terms=[(1, 4)], bias_id=5, act=None, oscale_id=3,
                     col=2 * DIM, wout=DIM),
                dict(terms=[(2, 6)], bias_id=7, act=None, oscale_id=None,
                     col=3 * DIM, wout=DIM),
            ],
            score=None, c_out=4 * DIM)
        return (out[:, 0:DIM], out[:, DIM:2 * DIM],
                out[:, 2 * DIM:3 * DIM], out[:, 3 * DIM:4 * DIM])

    x12, x21, x23, x32 = cross_pass(
        x_s1a, x_s2a, x_s3a,
        p["pool_s12_1"], p["unpool_s21_1"], p["pool_s23_1"], p["unpool_s32_1"])

    def tri_pass(parts, l1, l2, l3):
        w = [wt(l1), bias(l1), wt(l2), bias(l2), wt(l3), bias(l3)]
        out = _fused_pass(
            A, parts["arrays"], parts["scales"], w,
            parts=parts["groups"],
            outspecs=[
                dict(terms=[(0, 0)], bias_id=1, act="relu", oscale_id=None,
                     col=0, wout=DIM),
                dict(terms=[(1, 2)], bias_id=3, act="relu", oscale_id=None,
                     col=DIM, wout=DIM),
                dict(terms=[(2, 4)], bias_id=5, act="relu", oscale_id=None,
                     col=2 * DIM, wout=DIM),
            ],
            score=None, c_out=3 * DIM)
        return out[:, 0:DIM], out[:, DIM:2 * DIM], out[:, 2 * DIM:3 * DIM]

    p7 = dict(
        arrays=[x_s1a, x21, x_s1, x_s2a, x12, x32, x_s2, x_s3a, x23],
        scales=[sel1, sel2, m2],
        groups=[
            [(0, 1.0, None), (1, 1.0, None), (2, 1.0, None)],
            [(3, 1.0, 0), (4, 0.5, None), (5, 0.5, 0), (6, 1.0, 0)],
            [(7, 1.0, 1), (8, 1.0, None), (6, 1.0, 2)],
        ])
    x_s1b, x_s2b, x_s3b = tri_pass(p7, p["s1_l2"], p["s2_l2"], p["s3_l2"])

    x12b, x21b, x23b, x32b = cross_pass(
        x_s1b, x_s2b, x_s3b,
        p["pool_s12_2"], p["unpool_s21_2"], p["pool_s23_2"], p["unpool_s32_2"])

    p9 = dict(
        arrays=[x_s1b, x21b, x_s2b, x12b, x32b, x_s3b, x23b],
        scales=[sel1, sel2],
        groups=[
            [(0, 1.0, None), (1, 0.05, None)],
            [(2, 1.0, 0), (3, 0.025, None), (4, 0.025, 0)],
            [(5, 1.0, 1), (6, 0.05, None)],
        ])
    x_s1f, x_s2f, x_s3f = tri_pass(p9, p["s1_l3"], p["s2_l3"], p["s3_l3"])

    u32e = _fused_pass(
        A, [x_s3f], [sel2],
        [wt(p["unpool_s32_end"]), bias(p["unpool_s32_end"])],
        parts=[[(0, 1.0, 0)]],
        outspecs=[dict(terms=[(0, 0)], bias_id=1, act=None,
                       oscale_id=None, col=0, wout=DIM)],
        score=None, c_out=DIM)

    x_s2out = _fused_pass(
        A, [x_s2f, u32e, xdown2], [sel1],
        [wt(p["unpool_s21_end"]), bias(p["unpool_s21_end"])],
        parts=[[(0, 1.0, 0), (1, 1.0, 0), (2, 1.0, 0)]],
        outspecs=[dict(terms=[(0, 0)], bias_id=1, act=None,
                       oscale_id=None, col=0, wout=DIM)],
        score=None, c_out=DIM)

    wend = wt(p["end_gcn"])
    out = _fused_pass(
        A, [x_s1f, x_s2out], [],
        [wend[0:DIM, :], wend[DIM:2 * DIM, :], bias(p["end_gcn"])],
        parts=[[(0, 1.0, None)], [(1, 1.0, None)]],
        outspecs=[dict(terms=[(0, 0), (1, 1)], bias_id=2, act=None,
                       oscale_id=None, col=0, wout=256)],
        score=None, c_out=256)
    return out
